# baseline (speedup 1.0000x reference)
import jax
import jax.numpy as jnp
from jax import lax
from jax.experimental import pallas as pl
from jax.experimental.pallas import tpu as pltpu
from jax.experimental.pallas import tpu_sc as plsc

B = 4096
L = 50
NF = 5
CONT = 8
LARGE_DIM = 64
SMALL_DIM = 16
LANES = 16

NC = 2
NS = 16
W = NC * NS
UPW = B // W
SPW = UPW * L
CH = 128
NCH = SPW // CH
HC = CH // 2
GR = HC // LANES
OROWS = 56


def _sc_kernel(iidx_hbm, icont_hbm, uidx_hbm, ucont_hbm,
               ut0, ut1, ut2, ut3, ut4,
               it0, it1, it2, it3, it4,
               out_hbm,
               ridx_v, uidx_v, ucontv, u0c, u1c, u2c, u3c, u4c,
               rows0, rowsA, rowsB, icontv, accb, outb, sem0, semA, semB):
    wid = lax.axis_index("s") * NC + lax.axis_index("c")

    pltpu.sync_copy(uidx_hbm.at[wid], uidx_v)
    pltpu.sync_copy(ucont_hbm.at[wid], ucontv)

    for h in range(2):
        hs = pl.ds(h * HC, HC)
        pltpu.async_copy(ut0.at[uidx_v.at[0, hs]], rows0, sem0).wait()

        def u0fix(ub, _, h=h):
            lbv = uidx_v[NF, pl.ds(h * HC + ub * LANES, LANES)]
            for t in range(LANES):
                lr = ub * LANES + t
                lu = h * HC + lr
                uh = (lbv[t] & 1) * LARGE_DIM
                for c in range(4):
                    u0c[lu, pl.ds(c * 16, 16)] = (
                        rows0[lr, pl.ds(uh + c * 16, 16)])
            return 0

        lax.fori_loop(0, GR, u0fix, 0)

    user_small = ((ut1, u1c), (ut2, u2c), (ut3, u3c), (ut4, u4c))
    for f in range(1, NF):
        tab, ufc = user_small[f - 1]
        for h in range(2):
            hs = pl.ds(h * HC, HC)
            pltpu.async_copy(tab.at[uidx_v.at[f, hs]], rows0, sem0).wait()

            def ufix(ub, _, f=f, ufc=ufc, h=h):
                lbv = uidx_v[NF, pl.ds(h * HC + ub * LANES, LANES)]
                for t in range(LANES):
                    lr = ub * LANES + t
                    lu = h * HC + lr
                    sf = ((lbv[t] >> (3 * f)) & 7) * SMALL_DIM
                    ufc[lu] = rows0[lr, pl.ds(sf, 16)]
                return 0

            lax.fori_loop(0, GR, ufix, 0)

    small_tabs = (it1, it2, it3, it4)
    small_u = (u1c, u2c, u3c, u4c)

    def chunk_body(g, _):
        pltpu.sync_copy(iidx_hbm.at[wid, g], ridx_v)

        for h in range(2):
            hs = pl.ds(h * HC, HC)
            pltpu.sync_copy(icont_hbm.at[wid, g, pl.ds(h * 8, 8)], icontv)
            h0 = pltpu.async_copy(it0.at[ridx_v.at[0, hs]], rows0, sem0)
            h1 = pltpu.async_copy(it1.at[ridx_v.at[1, hs]], rowsA, semA)
            h0.wait()

            def p0(j16, _, h=h):
                jb = j16 * LANES
                lbv = ridx_v[NF, pl.ds(h * HC + jb, LANES)]
                for jj in range(LANES):
                    j = jb + jj
                    jc = h * HC + j
                    lu = (g * CH + jc) // L
                    ih = (lbv[jj] & 1) * LARGE_DIM
                    acc = (ucontv[lu >> 3, pl.ds((lu & 7) * 16, 16)]
                           * icontv[j >> 3, pl.ds((j & 7) * 16, 16)])
                    acc += u0c[lu, pl.ds(0, 16)] * rows0[j, pl.ds(ih, 16)]
                    acc += (u0c[lu, pl.ds(16, 16)]
                            * rows0[j, pl.ds(ih + 16, 16)])
                    acc += (u0c[lu, pl.ds(32, 16)]
                            * rows0[j, pl.ds(ih + 32, 16)])
                    acc += (u0c[lu, pl.ds(48, 16)]
                            * rows0[j, pl.ds(ih + 48, 16)])
                    accb[pl.ds(j * LANES, LANES)] = acc
                return 0

            lax.fori_loop(0, GR, p0, 0)

            def psmall(f, rbuf, h=h):
                ufc = small_u[f - 1]

                def body(j16, _):
                    jb = j16 * LANES
                    lbv = ridx_v[NF, pl.ds(h * HC + jb, LANES)]
                    for jj in range(LANES):
                        j = jb + jj
                        jc = h * HC + j
                        lu = (g * CH + jc) // L
                        sf = ((lbv[jj] >> (3 * f)) & 7) * SMALL_DIM
                        accb[pl.ds(j * LANES, LANES)] += (
                            ufc[lu] * rbuf[j, pl.ds(sf, 16)])
                    return 0

                lax.fori_loop(0, GR, body, 0)

            h1.wait()
            h2 = pltpu.async_copy(it2.at[ridx_v.at[2, hs]], rowsB, semB)
            psmall(1, rowsA)
            h2.wait()
            h3 = pltpu.async_copy(it3.at[ridx_v.at[3, hs]], rowsA, semA)
            psmall(2, rowsB)
            h3.wait()
            h4 = pltpu.async_copy(it4.at[ridx_v.at[4, hs]], rowsB, semB)
            psmall(3, rowsA)
            h4.wait()
            psmall(4, rowsB)

            def red_body(k, _, h=h):
                rowbase = (k * LANES + lax.iota(jnp.int32, LANES)) * LANES
                tot = jnp.zeros((LANES,), jnp.float32)
                for c in range(LANES):
                    tot = tot + plsc.load_gather(accb, [rowbase + c])
                outb[g & 7, pl.ds(h * HC + k * LANES, LANES)] = tot
                return 0

            lax.fori_loop(0, GR, red_body, 0)

        @pl.when(jnp.logical_or((g & 7) == 7, g == NCH - 1))
        def _():
            pltpu.sync_copy(outb, out_hbm.at[wid, pl.ds((g >> 3) * 8, 8)])

        return 0

    lax.fori_loop(0, NCH, chunk_body, 0)


@jax.jit
def kernel(user_sparse, item_sparse, user_cont, item_cont,
           user_t0, user_t1, user_t2, user_t3, user_t4,
           item_t0, item_t1, item_t2, item_t3, item_t4):
    ut0_r = user_t0.reshape(-1, 2 * LARGE_DIM)
    it0_r = item_t0.reshape(-1, 2 * LARGE_DIM)
    ut1_r = user_t1.reshape(-1, 8 * SMALL_DIM)
    ut2_r = user_t2.reshape(-1, 8 * SMALL_DIM)
    ut3_r = user_t3.reshape(-1, 8 * SMALL_DIM)
    ut4_r = user_t4.reshape(-1, 8 * SMALL_DIM)
    it1_r = item_t1.reshape(-1, 8 * SMALL_DIM)
    it2_r = item_t2.reshape(-1, 8 * SMALL_DIM)
    it3_r = item_t3.reshape(-1, 8 * SMALL_DIM)
    it4_r = item_t4.reshape(-1, 8 * SMALL_DIM)

    def prep_idx(raw):
        shifts = jnp.array([1, 3, 3, 3, 3], jnp.int32)
        gat = raw >> shifts
        low = (raw[..., 0] & 1)
        for f in range(1, NF):
            low = low | ((raw[..., f] & 7) << (3 * f))
        packed = jnp.concatenate(
            [gat, low[..., None],
             jnp.zeros(raw.shape[:-1] + (2,), jnp.int32)], axis=-1)
        return packed

    iidx = prep_idx(item_sparse.reshape(W, NCH, CH, NF)).transpose(0, 1, 3, 2)
    uidx = prep_idx(user_sparse.reshape(W, UPW, NF)).transpose(0, 2, 1)
    icont = jnp.pad(item_cont.reshape(B * L, CONT),
                    ((0, 0), (0, LANES - CONT)))
    icont = icont.reshape(W, NCH, LANES, 128)
    ucont = jnp.pad(user_cont, ((0, 0), (0, LANES - CONT)))
    ucont = ucont.reshape(W, LANES, 128)

    mesh = plsc.VectorSubcoreMesh(core_axis_name="c", subcore_axis_name="s")
    run = pl.kernel(
        _sc_kernel,
        mesh=mesh,
        compiler_params=pltpu.CompilerParams(needs_layout_passes=False),
        out_type=jax.ShapeDtypeStruct((W, OROWS, 128), jnp.float32),
        scratch_types=[
            pltpu.VMEM((8, CH), jnp.int32),
            pltpu.VMEM((8, UPW), jnp.int32),
            pltpu.VMEM((LANES, 128), jnp.float32),
            pltpu.VMEM((UPW, LARGE_DIM), jnp.float32),
            pltpu.VMEM((UPW, SMALL_DIM), jnp.float32),
            pltpu.VMEM((UPW, SMALL_DIM), jnp.float32),
            pltpu.VMEM((UPW, SMALL_DIM), jnp.float32),
            pltpu.VMEM((UPW, SMALL_DIM), jnp.float32),
            pltpu.VMEM((HC, 128), jnp.float32),
            pltpu.VMEM((HC, 128), jnp.float32),
            pltpu.VMEM((HC, 128), jnp.float32),
            pltpu.VMEM((8, 128), jnp.float32),
            pltpu.VMEM((HC * LANES,), jnp.float32),
            pltpu.VMEM((8, 128), jnp.float32),
            pltpu.SemaphoreType.DMA,
            pltpu.SemaphoreType.DMA,
            pltpu.SemaphoreType.DMA,
        ],
    )
    out = run(iidx, icont, uidx, ucont,
              ut0_r, ut1_r, ut2_r, ut3_r, ut4_r,
              it0_r, it1_r, it2_r, it3_r, it4_r)
    return out[:, :NCH, :].reshape(B, L)
